# Initial kernel scaffold; baseline (speedup 1.0000x reference)
import jax
import jax.numpy as jnp
from jax.experimental import pallas as pl


def kernel(x, edge_index, W, b):
    raise NotImplementedError("write your pallas kernel here")


# speedup vs baseline: 3.0962x; 3.0962x over previous
import functools

import jax
import jax.numpy as jnp
from jax import lax
from jax.experimental import pallas as pl
from jax.experimental.pallas import tpu as pltpu
from jax.experimental.pallas import tpu_sc as plsc

_N_NODES = 10000
_N_EDGES = 160000
_D = 256


_ROW_BLK = 1000


def _node_mm_body(x_ref, w_ref, b_ref, y_ref):
    w = w_ref[...]
    ws = w[:_D, :] + w[_D:, :]
    y_ref[...] = (
        jnp.dot(x_ref[...], ws, preferred_element_type=jnp.float32) + b_ref[...]
    )


def _node_transform(x, w, b2):
    grid = (_N_NODES // _ROW_BLK,)
    return pl.pallas_call(
        _node_mm_body,
        grid=grid,
        in_specs=[
            pl.BlockSpec((_ROW_BLK, _D), lambda i: (i, 0)),
            pl.BlockSpec((2 * _D, _D), lambda i: (0, 0)),
            pl.BlockSpec((1, _D), lambda i: (0, 0)),
        ],
        out_specs=pl.BlockSpec((_ROW_BLK, _D), lambda i: (i, 0)),
        out_shape=jax.ShapeDtypeStruct((_N_NODES, _D), jnp.float32),
    )(x, w, b2)


_NC = 2
_NS = 16
_NW = _NC * _NS
_EPW = _N_EDGES // _NW
_CHUNK = 40
_NCHUNK = _EPW // _CHUNK


def _edge_body(y_hbm, src_hbm, dst_hbm, out_hbm, src_v, dst_v, bufa, bufb,
               sema, semb):
    wid = lax.axis_index("s") * _NC + lax.axis_index("c")
    base = wid * _EPW
    pltpu.sync_copy(src_hbm.at[pl.ds(base, _EPW)], src_v)
    pltpu.sync_copy(dst_hbm.at[pl.ds(base, _EPW)], dst_v)

    def chunk_body(c, carry):
        off = c * _CHUNK
        ca = pltpu.async_copy(y_hbm.at[src_v.at[pl.ds(off, _CHUNK)]], bufa, sema)
        cb = pltpu.async_copy(y_hbm.at[dst_v.at[pl.ds(off, _CHUNK)]], bufb, semb)
        ca.wait()
        cb.wait()

        def row_body(r, rc):
            for j in range(_D // 16):
                sl = pl.ds(j * 16, 16)
                bufa[r, sl] = bufa[r, sl] + bufb[r, sl]
            return rc

        lax.fori_loop(0, _CHUNK, row_body, 0)
        pltpu.sync_copy(bufa, out_hbm.at[pl.ds(base + off, _CHUNK)])
        return carry

    lax.fori_loop(0, _NCHUNK, chunk_body, 0)


@functools.partial(
    pl.kernel,
    out_type=jax.ShapeDtypeStruct((_N_EDGES, _D), jnp.float32),
    mesh=plsc.VectorSubcoreMesh(core_axis_name="c", subcore_axis_name="s"),
    scratch_types=[
        pltpu.VMEM((_EPW,), jnp.int32),
        pltpu.VMEM((_EPW,), jnp.int32),
        pltpu.VMEM((_CHUNK, _D), jnp.float32),
        pltpu.VMEM((_CHUNK, _D), jnp.float32),
        pltpu.SemaphoreType.DMA,
        pltpu.SemaphoreType.DMA,
    ],
)
def _edge_gather_add(y_hbm, src_hbm, dst_hbm, out_hbm, src_v, dst_v,
                     bufa, bufb, sema, semb):
    _edge_body(y_hbm, src_hbm, dst_hbm, out_hbm, src_v, dst_v, bufa, bufb,
               sema, semb)


def kernel(x, edge_index, W, b):
    src = edge_index[0].astype(jnp.int32)
    dst = edge_index[1].astype(jnp.int32)
    y = _node_transform(x, W, b.reshape(1, _D))
    return _edge_gather_add(y, src, dst)


# speedup vs baseline: 5.0882x; 1.6434x over previous
import functools

import jax
import jax.numpy as jnp
from jax import lax
from jax.experimental import pallas as pl
from jax.experimental.pallas import tpu as pltpu
from jax.experimental.pallas import tpu_sc as plsc

_N_NODES = 10000
_N_EDGES = 160000
_D = 256


_ROW_BLK = 1000


def _node_mm_body(x_ref, w_ref, b_ref, y_ref):
    w = w_ref[...]
    ws = w[:_D, :] + w[_D:, :]
    y_ref[...] = (
        jnp.dot(x_ref[...], ws, preferred_element_type=jnp.float32) + b_ref[...]
    )


def _node_transform(x, w, b2):
    grid = (_N_NODES // _ROW_BLK,)
    return pl.pallas_call(
        _node_mm_body,
        grid=grid,
        in_specs=[
            pl.BlockSpec((_ROW_BLK, _D), lambda i: (i, 0)),
            pl.BlockSpec((2 * _D, _D), lambda i: (0, 0)),
            pl.BlockSpec((1, _D), lambda i: (0, 0)),
        ],
        out_specs=pl.BlockSpec((_ROW_BLK, _D), lambda i: (i, 0)),
        out_shape=jax.ShapeDtypeStruct((_N_NODES, _D), jnp.float32),
    )(x, w, b2)


_NC = 2
_NS = 16
_NW = _NC * _NS
_EPW = _N_EDGES // _NW
_CHUNK = 40
_NCHUNK = _EPW // _CHUNK


def _edge_body(y_hbm, src_hbm, dst_hbm, out_hbm, src_v, dst_v,
               bufa0, bufb0, bufa1, bufb1,
               sa0, sb0, sa1, sb1):
    wid = lax.axis_index("s") * _NC + lax.axis_index("c")
    base = wid * _EPW
    pltpu.sync_copy(src_hbm.at[pl.ds(base, _EPW)], src_v)
    pltpu.sync_copy(dst_hbm.at[pl.ds(base, _EPW)], dst_v)

    bufs = ((bufa0, bufb0, sa0, sb0), (bufa1, bufb1, sa1, sb1))

    def issue(c, slot):
        ba, bb, sa, sb = bufs[slot]
        off = c * _CHUNK
        pltpu.async_copy(y_hbm.at[src_v.at[pl.ds(off, _CHUNK)]], ba, sa)
        pltpu.async_copy(y_hbm.at[dst_v.at[pl.ds(off, _CHUNK)]], bb, sb)

    def process(c, slot):
        ba, bb, sa, sb = bufs[slot]
        pltpu.make_async_copy(y_hbm.at[src_v.at[pl.ds(0, _CHUNK)]], ba, sa).wait()
        pltpu.make_async_copy(y_hbm.at[dst_v.at[pl.ds(0, _CHUNK)]], bb, sb).wait()

        def row_body(r, rc):
            for j in range(_D // 16):
                sl = pl.ds(j * 16, 16)
                ba[r, sl] = ba[r, sl] + bb[r, sl]
            return rc

        lax.fori_loop(0, _CHUNK, row_body, 0)
        pltpu.sync_copy(ba, out_hbm.at[pl.ds(base + c * _CHUNK, _CHUNK)])

    issue(0, 0)

    def pair_body(g, carry):
        issue(2 * g + 1, 1)
        process(2 * g, 0)
        issue(2 * g + 2, 0)
        process(2 * g + 1, 1)
        return carry

    lax.fori_loop(0, (_NCHUNK - 1) // 2, pair_body, 0)
    process(_NCHUNK - 1, 0)


@functools.partial(
    pl.kernel,
    out_type=jax.ShapeDtypeStruct((_N_EDGES, _D), jnp.float32),
    mesh=plsc.VectorSubcoreMesh(core_axis_name="c", subcore_axis_name="s"),
    scratch_types=[
        pltpu.VMEM((_EPW,), jnp.int32),
        pltpu.VMEM((_EPW,), jnp.int32),
        pltpu.VMEM((_CHUNK, _D), jnp.float32),
        pltpu.VMEM((_CHUNK, _D), jnp.float32),
        pltpu.VMEM((_CHUNK, _D), jnp.float32),
        pltpu.VMEM((_CHUNK, _D), jnp.float32),
        pltpu.SemaphoreType.DMA,
        pltpu.SemaphoreType.DMA,
        pltpu.SemaphoreType.DMA,
        pltpu.SemaphoreType.DMA,
    ],
)
def _edge_gather_add(y_hbm, src_hbm, dst_hbm, out_hbm, src_v, dst_v,
                     bufa0, bufb0, bufa1, bufb1, sa0, sb0, sa1, sb1):
    _edge_body(y_hbm, src_hbm, dst_hbm, out_hbm, src_v, dst_v,
               bufa0, bufb0, bufa1, bufb1, sa0, sb0, sa1, sb1)


def kernel(x, edge_index, W, b):
    src = edge_index[0].astype(jnp.int32)
    dst = edge_index[1].astype(jnp.int32)
    y = _node_transform(x, W, b.reshape(1, _D))
    return _edge_gather_add(y, src, dst)
